# Initial kernel scaffold; baseline (speedup 1.0000x reference)
import jax
import jax.numpy as jnp
from jax.experimental import pallas as pl


def kernel(infeatn, adj_indices, adj_values, weight, bias):
    raise NotImplementedError("write your pallas kernel here")


# speedup vs baseline: 4.4943x; 4.4943x over previous
import functools

import jax
import jax.numpy as jnp
from jax import lax
from jax.experimental import pallas as pl
from jax.experimental.pallas import tpu as pltpu
from jax.experimental.pallas import tpu_sc as plsc

N = 10000
E = 320000
D = 128

NC = 2
NS = 16
L = 16
NW = NC * NS
EPW = E // NW
CH = 80
NCHUNK = EPW // CH
DG = D // L
ZROWS = 200
FW = 10
RPF = N // FW


def _mm_body(x_ref, w_ref, o_ref):
    o_ref[...] = jnp.dot(x_ref[...], w_ref[...],
                         preferred_element_type=jnp.float32)


def _matmul(x, w):
    mb = 2000
    return pl.pallas_call(
        _mm_body,
        grid=(N // mb,),
        in_specs=[pl.BlockSpec((mb, D), lambda i: (i, 0)),
                  pl.BlockSpec((D, D), lambda i: (0, 0))],
        out_specs=pl.BlockSpec((mb, D), lambda i: (i, 0)),
        out_shape=jax.ShapeDtypeStruct((N, D), jnp.float32),
    )(x, w)


def _agg_body(support, src, dst, vals, out,
              idx_v, dstidx_v, vals_v, rows_v, zbuf, acc, sem):
    c = lax.axis_index("c")
    s = lax.axis_index("s")
    base_w = (c * NS + s) * EPW
    row0 = s * RPF

    def zero_body(r, _):
        for j in range(DG):
            zbuf[r, pl.ds(j * L, L)] = jnp.zeros((L,), jnp.float32)
        return 0
    lax.fori_loop(0, ZROWS, zero_body, 0)

    @pl.when(s < FW)
    def _zero():
        for k in range(RPF // ZROWS):
            pltpu.sync_copy(zbuf, acc.at[pl.ds(row0 + k * ZROWS, ZROWS)])
    plsc.subcore_barrier()

    def chunk_body(t, _):
        base = base_w + t * CH
        pltpu.sync_copy(src.at[pl.ds(base, CH)], idx_v)
        pltpu.sync_copy(dst.at[pl.ds(base, CH)], dstidx_v)
        pltpu.sync_copy(vals.at[pl.ds(base, CH)], vals_v)
        pltpu.async_copy(support.at[idx_v], rows_v, sem).wait()
        dnums = lax.GatherDimensionNumbers(
            offset_dims=(), collapsed_slice_dims=(0,), start_index_map=(0,))
        for g in range(CH // L):
            vv = vals_v[pl.ds(g * L, L)]
            for e in range(L):
                splat = lax.gather(
                    vv, jnp.full((L, 1), e, jnp.int32), dnums, (1,),
                    mode=lax.GatherScatterMode.PROMISE_IN_BOUNDS)
                r = g * L + e
                for j in range(DG):
                    rows_v[r, pl.ds(j * L, L)] = (
                        rows_v[r, pl.ds(j * L, L)] * splat)
        pltpu.sync_copy(rows_v, acc.at[dstidx_v], add=True)
        return 0

    lax.fori_loop(0, NCHUNK, chunk_body, 0)
    plsc.subcore_barrier()

    @pl.when(s < FW)
    def _flush():
        pltpu.sync_copy(acc.at[pl.ds(row0, RPF)],
                        out.at[c, pl.ds(row0, RPF)])


def _aggregate(support, src, dst, vals):
    mesh = plsc.VectorSubcoreMesh(core_axis_name="c", subcore_axis_name="s")
    f = functools.partial(
        pl.kernel,
        out_type=jax.ShapeDtypeStruct((NC, N, D), jnp.float32),
        mesh=mesh,
        scratch_types=[
            pltpu.VMEM((CH,), jnp.int32),
            pltpu.VMEM((CH,), jnp.int32),
            pltpu.VMEM((CH,), jnp.float32),
            pltpu.VMEM((CH, D), jnp.float32),
            pltpu.VMEM((ZROWS, D), jnp.float32),
            pltpu.VMEM_SHARED((N, D), jnp.float32),
            pltpu.SemaphoreType.DMA,
        ],
    )(_agg_body)
    return f(support, src, dst, vals)


def _comb_body(p_ref, b_ref, o_ref):
    o_ref[...] = p_ref[0] + p_ref[1] + b_ref[...]


def _combine(partials, bias2d):
    mb = 2000
    return pl.pallas_call(
        _comb_body,
        grid=(N // mb,),
        in_specs=[pl.BlockSpec((NC, mb, D), lambda i: (0, i, 0)),
                  pl.BlockSpec((1, D), lambda i: (0, 0))],
        out_specs=pl.BlockSpec((mb, D), lambda i: (i, 0)),
        out_shape=jax.ShapeDtypeStruct((N, D), jnp.float32),
    )(partials, bias2d)


def kernel(infeatn, adj_indices, adj_values, weight, bias):
    support = _matmul(infeatn, weight)
    dst = adj_indices[0]
    src = adj_indices[1]
    partials = _aggregate(support, src, dst, adj_values)
    return _combine(partials, bias.reshape(1, D))


# speedup vs baseline: 12.8146x; 2.8513x over previous
import functools

import jax
import jax.numpy as jnp
from jax import lax
from jax.experimental import pallas as pl
from jax.experimental.pallas import tpu as pltpu
from jax.experimental.pallas import tpu_sc as plsc

N = 10000
E = 320000
D = 128

NC = 2
NS = 16
L = 16
NW = NC * NS
EPW = E // NW
CH = 80
NCHUNK = EPW // CH
DG = D // L
FW = 10
RPF = N // FW


def _mm_body(p_ref, w_ref, b_ref, o_ref):
    o_ref[...] = jnp.dot(p_ref[0] + p_ref[1], w_ref[...],
                         preferred_element_type=jnp.float32) + b_ref[...]


def _matmul_combine(partials, w, bias2d):
    mb = 2000
    return pl.pallas_call(
        _mm_body,
        grid=(N // mb,),
        in_specs=[pl.BlockSpec((NC, mb, D), lambda i: (0, i, 0)),
                  pl.BlockSpec((D, D), lambda i: (0, 0)),
                  pl.BlockSpec((1, D), lambda i: (0, 0))],
        out_specs=pl.BlockSpec((mb, D), lambda i: (i, 0)),
        out_shape=jax.ShapeDtypeStruct((N, D), jnp.float32),
    )(partials, w, bias2d)


def _agg_body(support, src, dst, vals, out,
              src_v, d0, d1, d2, v0, v1, v2, r0, r1, r2, acc,
              g0, g1, g2, i0, i1, i2, ss0, ss1, ss2):
    c = lax.axis_index("c")
    s = lax.axis_index("s")
    w = c * NS + s
    ebase = w * EPW
    row0 = s * RPF

    pltpu.async_copy(src.at[w], src_v, g0)

    def zero_body(r, _):
        for j in range(DG):
            r0[r, pl.ds(j * L, L)] = jnp.zeros((L,), jnp.float32)
        return 0
    lax.fori_loop(0, CH, zero_body, 0)

    @pl.when(s < FW)
    def _zero():
        for k in range(RPF // CH):
            pltpu.async_copy(r0, acc.at[pl.ds(row0 + k * CH, CH)], i1)
        pltpu.async_copy(r0.at[pl.ds(0, RPF % CH)],
                         acc.at[pl.ds(row0 + (RPF // CH) * CH, RPF % CH)],
                         i1)
        for k in range(RPF // CH):
            pltpu.make_async_copy(r0, acc.at[pl.ds(row0, CH)], i1).wait()
        pltpu.make_async_copy(r0.at[pl.ds(0, RPF % CH)],
                              acc.at[pl.ds(row0, RPF % CH)], i1).wait()

    pltpu.make_async_copy(src.at[w], src_v, g0).wait()
    plsc.subcore_barrier()

    rows = (r0, r1, r2)
    gsem = (g0, g1, g2)
    sssem = (ss0, ss1, ss2)
    dbufs = (d0, d1, d2)
    vbufs = (v0, v1, v2)
    isem = (i0, i1, i2)

    def fetch(t, k):
        pltpu.async_copy(dst.at[pl.ds(ebase + t * CH, CH)], dbufs[k], isem[k])
        pltpu.async_copy(vals.at[pl.ds(ebase + t * CH, CH)], vbufs[k], isem[k])

    def wait_fetch(k):
        pltpu.make_async_copy(dst.at[pl.ds(ebase, CH)], dbufs[k],
                              isem[k]).wait()
        pltpu.make_async_copy(vals.at[pl.ds(ebase, CH)], vbufs[k],
                              isem[k]).wait()

    def issue(t, k):
        pltpu.async_copy(support.at[src_v.at[t]], rows[k], gsem[k])

    def drain(k):
        pltpu.make_async_copy(support.at[src_v.at[0]], rows[k],
                              gsem[k]).wait()

    def sc_issue(k):
        pltpu.async_copy(rows[k], acc.at[dbufs[k]], sssem[k], add=True)

    def sc_wait(k):
        pltpu.make_async_copy(rows[k], acc.at[dbufs[k]], sssem[k]).wait()

    dnums = lax.GatherDimensionNumbers(
        offset_dims=(), collapsed_slice_dims=(0,), start_index_map=(0,))

    def scale(k):
        buf = rows[k]
        vbuf = vbufs[k]

        def g_body(g, _):
            vv = vbuf[pl.ds(g * L, L)]
            for e in range(L):
                splat = lax.gather(
                    vv, jnp.full((L, 1), e, jnp.int32), dnums, (1,),
                    mode=lax.GatherScatterMode.PROMISE_IN_BOUNDS)
                r = g * L + e
                for j in range(DG):
                    buf[r, pl.ds(j * L, L)] = buf[r, pl.ds(j * L, L)] * splat
            return 0

        lax.fori_loop(0, CH // L, g_body, 0)

    def step(k, first=False):
        drain(k)
        wait_fetch(k)
        scale(k)
        if not first:
            sc_wait((k + 2) % 3)
        sc_issue(k)

    fetch(0, 0)
    fetch(1, 1)
    issue(0, 0)
    issue(1, 1)
    step(0, first=True)
    fetch(2, 2)
    issue(2, 2)

    def body(u, _):
        t1 = 3 * u + 1
        step(1)
        fetch(t1 + 2, 0)
        issue(t1 + 2, 0)
        step(2)
        fetch(t1 + 3, 1)
        issue(t1 + 3, 1)
        step(0)

        @pl.when(t1 + 4 < NCHUNK)
        def _():
            fetch(t1 + 4, 2)
            issue(t1 + 4, 2)
        return 0

    lax.fori_loop(0, (NCHUNK - 2) // 3, body, 0)
    step(1)
    sc_wait(1)
    plsc.subcore_barrier()

    @pl.when(s < FW)
    def _flush():
        pltpu.sync_copy(acc.at[pl.ds(row0, RPF)],
                        out.at[c, pl.ds(row0, RPF)])


def _aggregate(support, src3d, dst1d, vals1d):
    mesh = plsc.VectorSubcoreMesh(core_axis_name="c", subcore_axis_name="s")
    f = functools.partial(
        pl.kernel,
        out_type=jax.ShapeDtypeStruct((NC, N, D), jnp.float32),
        mesh=mesh,
        scratch_types=[
            pltpu.VMEM((NCHUNK, CH), jnp.int32),
            pltpu.VMEM((CH,), jnp.int32),
            pltpu.VMEM((CH,), jnp.int32),
            pltpu.VMEM((CH,), jnp.int32),
            pltpu.VMEM((CH,), jnp.float32),
            pltpu.VMEM((CH,), jnp.float32),
            pltpu.VMEM((CH,), jnp.float32),
            pltpu.VMEM((CH, D), jnp.float32),
            pltpu.VMEM((CH, D), jnp.float32),
            pltpu.VMEM((CH, D), jnp.float32),
            pltpu.VMEM_SHARED((N, D), jnp.float32),
            pltpu.SemaphoreType.DMA,
            pltpu.SemaphoreType.DMA,
            pltpu.SemaphoreType.DMA,
            pltpu.SemaphoreType.DMA,
            pltpu.SemaphoreType.DMA,
            pltpu.SemaphoreType.DMA,
            pltpu.SemaphoreType.DMA,
            pltpu.SemaphoreType.DMA,
            pltpu.SemaphoreType.DMA,
        ],
    )(_agg_body)
    return f(support, src3d, dst1d, vals1d)


def kernel(infeatn, adj_indices, adj_values, weight, bias):
    dst1d = adj_indices[0]
    src3d = adj_indices[1].reshape(NW, NCHUNK, CH)
    partials = _aggregate(infeatn, src3d, dst1d, adj_values)
    return _matmul_combine(partials, weight, bias.reshape(1, D))


# speedup vs baseline: 13.1219x; 1.0240x over previous
import functools

import jax
import jax.numpy as jnp
from jax import lax
from jax.experimental import pallas as pl
from jax.experimental.pallas import tpu as pltpu
from jax.experimental.pallas import tpu_sc as plsc

N = 10000
E = 320000
D = 128

NC = 2
NS = 16
L = 16
NW = NC * NS
EPW = E // NW
CH = 80
NCHUNK = EPW // CH
DG = D // L
FW = 10
RPF = N // FW


def _mm_body(p_ref, w_ref, b_ref, o_ref):
    o_ref[...] = jnp.dot(p_ref[0] + p_ref[1], w_ref[...],
                         preferred_element_type=jnp.float32,
                         precision=lax.Precision.HIGHEST) + b_ref[...]


def _matmul_combine(partials, w, bias2d):
    mb = 2000
    return pl.pallas_call(
        _mm_body,
        grid=(N // mb,),
        in_specs=[pl.BlockSpec((NC, mb, D), lambda i: (0, i, 0)),
                  pl.BlockSpec((D, D), lambda i: (0, 0)),
                  pl.BlockSpec((1, D), lambda i: (0, 0))],
        out_specs=pl.BlockSpec((mb, D), lambda i: (i, 0)),
        out_shape=jax.ShapeDtypeStruct((N, D), jnp.float32),
    )(partials, w, bias2d)


NB = 4


def _agg_body(feat, src, dst, vals, out,
              s0, s1, s2, s3, d0, d1, d2, d3, v0, v1, v2, v3,
              r0, r1, r2, r3, acc,
              gs0, gs1, gs2, gs3, fs0, fs1, fs2, fs3,
              es0, es1, es2, es3, ws0, ws1, ws2, ws3):
    c = lax.axis_index("c")
    s = lax.axis_index("s")
    w = c * NS + s
    ebase = w * EPW
    row0 = s * RPF

    srcb = (s0, s1, s2, s3)
    dbufs = (d0, d1, d2, d3)
    vbufs = (v0, v1, v2, v3)
    rows = (r0, r1, r2, r3)
    gsem = (gs0, gs1, gs2, gs3)
    fsem = (fs0, fs1, fs2, fs3)
    esem = (es0, es1, es2, es3)
    sssem = (ws0, ws1, ws2, ws3)

    def zero_body(r, _):
        for j in range(DG):
            r0[r, pl.ds(j * L, L)] = jnp.zeros((L,), jnp.float32)
        return 0
    lax.fori_loop(0, CH, zero_body, 0)

    @pl.when(s < FW)
    def _zero():
        for k in range(RPF // CH):
            pltpu.async_copy(r0, acc.at[pl.ds(row0 + k * CH, CH)], gs0)
        pltpu.async_copy(r0.at[pl.ds(0, RPF % CH)],
                         acc.at[pl.ds(row0 + (RPF // CH) * CH, RPF % CH)],
                         gs0)
        for k in range(RPF // CH):
            pltpu.make_async_copy(r0, acc.at[pl.ds(row0, CH)], gs0).wait()
        pltpu.make_async_copy(r0.at[pl.ds(0, RPF % CH)],
                              acc.at[pl.ds(row0, RPF % CH)], gs0).wait()
    plsc.subcore_barrier()

    def fetch_src(t, k):
        pltpu.async_copy(src.at[pl.ds(ebase + t * CH, CH)], srcb[k], esem[k])

    def wait_src(k):
        pltpu.make_async_copy(src.at[pl.ds(ebase, CH)], srcb[k],
                              esem[k]).wait()

    def fetch_dv(t, k):
        pltpu.async_copy(dst.at[pl.ds(ebase + t * CH, CH)], dbufs[k],
                         fsem[k])
        pltpu.async_copy(vals.at[pl.ds(ebase + t * CH, CH)], vbufs[k],
                         fsem[k])

    def wait_dv(k):
        pltpu.make_async_copy(dst.at[pl.ds(ebase, CH)], dbufs[k],
                              fsem[k]).wait()
        pltpu.make_async_copy(vals.at[pl.ds(ebase, CH)], vbufs[k],
                              fsem[k]).wait()

    def issue(k):
        pltpu.async_copy(feat.at[srcb[k]], rows[k], gsem[k])

    def drain(k):
        pltpu.make_async_copy(feat.at[srcb[0]], rows[k], gsem[k]).wait()

    def sc_issue(k):
        pltpu.async_copy(rows[k], acc.at[dbufs[k]], sssem[k], add=True)

    def sc_wait(k):
        pltpu.make_async_copy(rows[k], acc.at[dbufs[k]], sssem[k]).wait()

    dnums = lax.GatherDimensionNumbers(
        offset_dims=(), collapsed_slice_dims=(0,), start_index_map=(0,))

    def scale(k):
        buf = rows[k]
        vbuf = vbufs[k]

        def g_body(g, _):
            vv = vbuf[pl.ds(g * L, L)]
            for e in range(L):
                splat = lax.gather(
                    vv, jnp.full((L, 1), e, jnp.int32), dnums, (1,),
                    mode=lax.GatherScatterMode.PROMISE_IN_BOUNDS)
                r = g * L + e
                for j in range(DG):
                    buf[r, pl.ds(j * L, L)] = buf[r, pl.ds(j * L, L)] * splat
            return 0

        lax.fori_loop(0, CH // L, g_body, 0)

    def step(t, k, first=False, more3=True, more4=True):
        drain(k)
        if more4:
            fetch_src(t + NB, k)
        wait_dv(k)
        scale(k)
        if not first:
            sc_wait((k + 3) % NB)
            if more3:
                fetch_dv(t + 3, (k + 3) % NB)
        sc_issue(k)
        if more3:
            wait_src((k + 3) % NB)
            issue((k + 3) % NB)

    for t in range(NB):
        fetch_src(t, t)
        fetch_dv(t, t)
    for t in range(3):
        wait_src(t)
        issue(t)
    step(0, 0, first=True)

    def body(u, _):
        t1 = NB * u + 1
        step(t1, 1)
        step(t1 + 1, 2)
        step(t1 + 2, 3)
        step(t1 + 3, 0)
        return 0

    lax.fori_loop(0, (NCHUNK - 1) // NB - 1, body, 0)
    t1 = NCHUNK - NB
    step(t1, 1, more4=False)
    step(t1 + 1, 2, more3=False, more4=False)
    step(t1 + 2, 3, more3=False, more4=False)
    step(t1 + 3, 0, more3=False, more4=False)
    sc_wait(0)
    plsc.subcore_barrier()

    @pl.when(s < FW)
    def _flush():
        pltpu.sync_copy(acc.at[pl.ds(row0, RPF)],
                        out.at[c, pl.ds(row0, RPF)])


def _aggregate(feat, src1d, dst1d, vals1d):
    mesh = plsc.VectorSubcoreMesh(core_axis_name="c", subcore_axis_name="s")
    f = functools.partial(
        pl.kernel,
        out_type=jax.ShapeDtypeStruct((NC, N, D), jnp.float32),
        mesh=mesh,
        scratch_types=(
            [pltpu.VMEM((CH,), jnp.int32) for _ in range(NB)]
            + [pltpu.VMEM((CH,), jnp.int32) for _ in range(NB)]
            + [pltpu.VMEM((CH,), jnp.float32) for _ in range(NB)]
            + [pltpu.VMEM((CH, D), jnp.float32) for _ in range(NB)]
            + [pltpu.VMEM_SHARED((N, D), jnp.float32)]
            + [pltpu.SemaphoreType.DMA for _ in range(4 * NB)]
        ),
    )(_agg_body)
    return f(feat, src1d, dst1d, vals1d)


def kernel(infeatn, adj_indices, adj_values, weight, bias):
    partials = _aggregate(infeatn, adj_indices[1], adj_indices[0],
                          adj_values)
    return _matmul_combine(partials, weight, bias.reshape(1, D))
